# baseline (speedup 1.0000x reference)
import jax
import jax.numpy as jnp
from jax.experimental import pallas as pl
from jax.experimental.pallas import tpu as pltpu

N = 131072
K = 64
M = 16
R = 4096
NB = N // R
ALPHA = 0.7
BETA = 1.5


def _p1_body(true_ref, emb_ref, sum_ref, cnt_ref, cntrow_ref):
    step = pl.program_id(0)

    @pl.when(step == 0)
    def _():
        sum_ref[...] = jnp.zeros_like(sum_ref)
        cnt_ref[...] = jnp.zeros_like(cnt_ref)
        cntrow_ref[...] = jnp.zeros_like(cntrow_ref)

    lab = true_ref[...]
    oh_t = (lab == jax.lax.broadcasted_iota(jnp.int32, (M, 1), 0)
            ).astype(jnp.float32)
    sum_ref[...] += jax.lax.dot_general(
        oh_t, emb_ref[0], (((1,), (0,)), ((), ())),
        preferred_element_type=jnp.float32)
    ones = jnp.ones((R, 8), jnp.float32)
    cnt_ref[...] += jax.lax.dot_general(
        oh_t, ones, (((1,), (0,)), ((), ())),
        preferred_element_type=jnp.float32)
    cntrow_ref[...] += jax.lax.dot_general(
        jnp.ones((1, R), jnp.float32), oh_t, (((1,), (1,)), ((), ())),
        preferred_element_type=jnp.float32)


def _p2_body(true_l_ref, emb_ref, pts_ref, sum_ref, cnt_ref, cntrow_ref,
             out_ref, mean_s, wrow_s, acc_s):
    step = pl.program_id(0)

    @pl.when(step == 0)
    def _():
        mean_s[...] = sum_ref[...] / cnt_ref[:, :1]
        lane_ids = jax.lax.broadcasted_iota(jnp.int32, (1, M), 1)
        labmask = (lane_ids >= 1).astype(jnp.float32)
        wrow_s[...] = labmask / cntrow_ref[...]
        acc_s[...] = jnp.zeros_like(acc_s)
        out_ref[...] = jnp.zeros_like(out_ref)

    lab = true_l_ref[...]
    oh_t = (lab == jax.lax.broadcasted_iota(jnp.int32, (M, 1), 0)
            ).astype(jnp.float32)

    emb = emb_ref[0]
    dt = jax.lax.dot_general(
        mean_s[...], emb, (((1,), (1,)), ((), ())),
        preferred_element_type=jnp.float32)
    dot_own = jnp.sum(oh_t * dt, axis=0, keepdims=True)
    sq = emb * emb
    e2 = jax.lax.dot_general(
        jnp.ones((1, K), jnp.float32), sq, (((1,), (1,)), ((), ())),
        preferred_element_type=jnp.float32)
    m2 = jnp.sum(mean_s[...] * mean_s[...], axis=1, keepdims=True)
    m2_own = jnp.sum(oh_t * m2, axis=0, keepdims=True)
    d2 = jnp.maximum(e2 - 2.0 * dot_own + m2_own, 0.0)
    d = jnp.sqrt(d2)

    pts = pts_ref[0]
    psq = jax.lax.dot_general(
        jnp.ones((1, 3), jnp.float32),
        pts * pts, (((1,), (1,)), ((), ())),
        preferred_element_type=jnp.float32)
    g = jax.nn.sigmoid(jnp.sqrt(psq))

    w = jax.lax.dot_general(
        wrow_s[...], oh_t, (((1,), (0,)), ((), ())),
        preferred_element_type=jnp.float32)
    hinge = jnp.maximum(d - ALPHA, 0.0)
    acc_s[...] += g * hinge * hinge * w

    @pl.when(step == NB - 1)
    def _():
        intra = jnp.sum(acc_s[...])

        m = mean_s[...]
        gram = jax.lax.dot_general(
            m, m, (((1,), (1,)), ((), ())),
            preferred_element_type=jnp.float32)
        ii = jax.lax.broadcasted_iota(jnp.int32, (M, M), 0)
        jj = jax.lax.broadcasted_iota(jnp.int32, (M, M), 1)
        diag = (ii == jj).astype(jnp.float32)
        nrm_col = jnp.sum(gram * diag, axis=1, keepdims=True)
        nrm_row = jnp.sum(gram * diag, axis=0, keepdims=True)
        d2p = jnp.maximum(nrm_col + nrm_row - 2.0 * gram, 0.0)
        dp = jnp.sqrt(d2p)
        hp = jnp.maximum(BETA - dp, 0.0)
        offdiag = ((ii != jj) & (ii >= 1) & (jj >= 1)).astype(jnp.float32)
        inter = jnp.sum(hp * hp * offdiag)

        val = intra / M + inter / (M * (M - 1))
        out_ref[...] = val.reshape(1, 1)


def kernel(points, true, embedding):
    seg_sum, cnt, cntrow = pl.pallas_call(
        _p1_body,
        grid=(NB,),
        in_specs=[
            pl.BlockSpec((1, R), lambda i: (0, i)),
            pl.BlockSpec((1, R, K), lambda i: (0, i, 0)),
        ],
        out_specs=[
            pl.BlockSpec((M, K), lambda i: (0, 0)),
            pl.BlockSpec((M, 8), lambda i: (0, 0)),
            pl.BlockSpec((1, M), lambda i: (0, 0)),
        ],
        out_shape=[
            jax.ShapeDtypeStruct((M, K), jnp.float32),
            jax.ShapeDtypeStruct((M, 8), jnp.float32),
            jax.ShapeDtypeStruct((1, M), jnp.float32),
        ],
    )(true, embedding)

    out = pl.pallas_call(
        _p2_body,
        grid=(NB,),
        in_specs=[
            pl.BlockSpec((1, R), lambda i: (0, i)),
            pl.BlockSpec((1, R, K), lambda i: (0, i, 0)),
            pl.BlockSpec((1, R, 3), lambda i: (0, i, 0)),
            pl.BlockSpec((M, K), lambda i: (0, 0)),
            pl.BlockSpec((M, 8), lambda i: (0, 0)),
            pl.BlockSpec((1, M), lambda i: (0, 0)),
        ],
        out_specs=pl.BlockSpec((1, 1), lambda i: (0, 0)),
        out_shape=jax.ShapeDtypeStruct((1, 1), jnp.float32),
        scratch_shapes=[
            pltpu.VMEM((M, K), jnp.float32),
            pltpu.VMEM((1, M), jnp.float32),
            pltpu.VMEM((1, R), jnp.float32),
        ],
    )(true, embedding, points, seg_sum, cnt, cntrow)

    return out.reshape(1)


# speedup vs baseline: 1.1455x; 1.1455x over previous
import jax
import jax.numpy as jnp
from jax.experimental import pallas as pl
from jax.experimental.pallas import tpu as pltpu

N = 131072
K = 64
M = 16
R = 16384
NB = N // R
ALPHA = 0.7
BETA = 1.5


def _p1_body(true_ref, emb_ref, sum_ref, cnt_ref, cntrow_ref):
    step = pl.program_id(0)

    @pl.when(step == 0)
    def _():
        sum_ref[...] = jnp.zeros_like(sum_ref)
        cnt_ref[...] = jnp.zeros_like(cnt_ref)
        cntrow_ref[...] = jnp.zeros_like(cntrow_ref)

    lab = true_ref[...]
    oh_t = (lab == jax.lax.broadcasted_iota(jnp.int32, (M, 1), 0)
            ).astype(jnp.float32)
    sum_ref[...] += jax.lax.dot_general(
        oh_t, emb_ref[0], (((1,), (0,)), ((), ())),
        preferred_element_type=jnp.float32)
    ones = jnp.ones((R, 8), jnp.float32)
    cnt_ref[...] += jax.lax.dot_general(
        oh_t, ones, (((1,), (0,)), ((), ())),
        preferred_element_type=jnp.float32)
    cntrow_ref[...] += jax.lax.dot_general(
        jnp.ones((1, R), jnp.float32), oh_t, (((1,), (1,)), ((), ())),
        preferred_element_type=jnp.float32)


def _p2_body(true_l_ref, emb_ref, pts_ref, sum_ref, cnt_ref, cntrow_ref,
             out_ref, mean_s, wrow_s, acc_s):
    step = pl.program_id(0)

    @pl.when(step == 0)
    def _():
        mean_s[...] = sum_ref[...] / cnt_ref[:, :1]
        lane_ids = jax.lax.broadcasted_iota(jnp.int32, (1, M), 1)
        labmask = (lane_ids >= 1).astype(jnp.float32)
        wrow_s[...] = labmask / cntrow_ref[...]
        acc_s[...] = jnp.zeros_like(acc_s)
        out_ref[...] = jnp.zeros_like(out_ref)

    lab = true_l_ref[...]
    oh_t = (lab == jax.lax.broadcasted_iota(jnp.int32, (M, 1), 0)
            ).astype(jnp.float32)

    emb = emb_ref[0]
    dt = jax.lax.dot_general(
        mean_s[...], emb, (((1,), (1,)), ((), ())),
        preferred_element_type=jnp.float32)
    dot_own = jnp.sum(oh_t * dt, axis=0, keepdims=True)
    sq = emb * emb
    e2 = jax.lax.dot_general(
        jnp.ones((1, K), jnp.float32), sq, (((1,), (1,)), ((), ())),
        preferred_element_type=jnp.float32)
    m2 = jnp.sum(mean_s[...] * mean_s[...], axis=1, keepdims=True)
    m2_own = jnp.sum(oh_t * m2, axis=0, keepdims=True)
    d2 = jnp.maximum(e2 - 2.0 * dot_own + m2_own, 0.0)
    d = jnp.sqrt(d2)

    pts = pts_ref[0]
    psq = jax.lax.dot_general(
        jnp.ones((1, 3), jnp.float32),
        pts * pts, (((1,), (1,)), ((), ())),
        preferred_element_type=jnp.float32)
    g = jax.nn.sigmoid(jnp.sqrt(psq))

    w = jax.lax.dot_general(
        wrow_s[...], oh_t, (((1,), (0,)), ((), ())),
        preferred_element_type=jnp.float32)
    hinge = jnp.maximum(d - ALPHA, 0.0)
    acc_s[...] += g * hinge * hinge * w

    @pl.when(step == NB - 1)
    def _():
        intra = jnp.sum(acc_s[...])

        m = mean_s[...]
        gram = jax.lax.dot_general(
            m, m, (((1,), (1,)), ((), ())),
            preferred_element_type=jnp.float32)
        ii = jax.lax.broadcasted_iota(jnp.int32, (M, M), 0)
        jj = jax.lax.broadcasted_iota(jnp.int32, (M, M), 1)
        diag = (ii == jj).astype(jnp.float32)
        nrm_col = jnp.sum(gram * diag, axis=1, keepdims=True)
        nrm_row = jnp.sum(gram * diag, axis=0, keepdims=True)
        d2p = jnp.maximum(nrm_col + nrm_row - 2.0 * gram, 0.0)
        dp = jnp.sqrt(d2p)
        hp = jnp.maximum(BETA - dp, 0.0)
        offdiag = ((ii != jj) & (ii >= 1) & (jj >= 1)).astype(jnp.float32)
        inter = jnp.sum(hp * hp * offdiag)

        val = intra / M + inter / (M * (M - 1))
        out_ref[...] = val.reshape(1, 1)


def kernel(points, true, embedding):
    seg_sum, cnt, cntrow = pl.pallas_call(
        _p1_body,
        grid=(NB,),
        in_specs=[
            pl.BlockSpec((1, R), lambda i: (0, i)),
            pl.BlockSpec((1, R, K), lambda i: (0, i, 0)),
        ],
        out_specs=[
            pl.BlockSpec((M, K), lambda i: (0, 0)),
            pl.BlockSpec((M, 8), lambda i: (0, 0)),
            pl.BlockSpec((1, M), lambda i: (0, 0)),
        ],
        out_shape=[
            jax.ShapeDtypeStruct((M, K), jnp.float32),
            jax.ShapeDtypeStruct((M, 8), jnp.float32),
            jax.ShapeDtypeStruct((1, M), jnp.float32),
        ],
    )(true, embedding)

    out = pl.pallas_call(
        _p2_body,
        grid=(NB,),
        in_specs=[
            pl.BlockSpec((1, R), lambda i: (0, i)),
            pl.BlockSpec((1, R, K), lambda i: (0, i, 0)),
            pl.BlockSpec((1, R, 3), lambda i: (0, i, 0)),
            pl.BlockSpec((M, K), lambda i: (0, 0)),
            pl.BlockSpec((M, 8), lambda i: (0, 0)),
            pl.BlockSpec((1, M), lambda i: (0, 0)),
        ],
        out_specs=pl.BlockSpec((1, 1), lambda i: (0, 0)),
        out_shape=jax.ShapeDtypeStruct((1, 1), jnp.float32),
        scratch_shapes=[
            pltpu.VMEM((M, K), jnp.float32),
            pltpu.VMEM((1, M), jnp.float32),
            pltpu.VMEM((1, R), jnp.float32),
        ],
    )(true, embedding, points, seg_sum, cnt, cntrow)

    return out.reshape(1)
